# baseline (speedup 1.0000x reference)
import functools

import jax
import jax.numpy as jnp
import numpy as np
from jax import lax
from jax.experimental import pallas as pl
from jax.experimental.pallas import tpu as pltpu
from jax.experimental.pallas import tpu_sc as plsc

B, L, HIDDEN = 1024, 200, 128
N = B * L
NC, NS = 2, 16
NW = NC * NS
PER_W = N // NW
S = 5
PER_WS = PER_W // S
CHUNK = 128
NCH_S = PER_WS // CHUNK
NSL = NW * PER_WS
TBLK = PER_WS
NBLK_TOTAL = N // TBLK


def _pack_rows(rows_f32, rows_i32):
    def prow(r, carry):
        for p in range(4):
            a = lax.bitcast_convert_type(
                rows_f32[r, pl.ds(p * 32, 16)], jnp.int32) + 32768
            bb = lax.bitcast_convert_type(
                rows_f32[r, pl.ds(p * 32 + 16, 16)], jnp.int32) + 32768
            lo = lax.shift_right_logical(a, 16)
            hi = lax.bitwise_and(bb, jnp.int32(-65536))
            rows_i32[r, pl.ds(p * 16, 16)] = lax.bitwise_or(lo, hi)
        return carry

    lax.fori_loop(0, CHUNK, prow, 0)


_pm_nat = np.empty((128,), np.int32)
for _m in range(128):
    _g, _o = divmod(_m, 32)
    _i, _r = divmod(_o, 2)
    _pm_nat[_m] = 32 * _g + _i + 16 * _r
_PM_NP = (_pm_nat[:, None] == np.arange(128)[None, :]).astype(np.float32)


def _sc_word_gather(word_table, ids_grp):
    @functools.partial(
        pl.kernel,
        mesh=plsc.VectorSubcoreMesh(core_axis_name="c", subcore_axis_name="s"),
        out_type=jax.ShapeDtypeStruct((NSL, HIDDEN // 2), jnp.int32),
        scratch_types=[
            pltpu.VMEM((NCH_S, CHUNK), jnp.int32),
            pltpu.VMEM((2, CHUNK, HIDDEN), jnp.float32),
            pltpu.VMEM((2, CHUNK, HIDDEN // 2), jnp.int32),
            pltpu.SemaphoreType.DMA,
            pltpu.SemaphoreType.DMA,
        ],
    )
    def k(table_hbm, idx_hbm, out_hbm, idx_v, rows_v, rowsb_v, sem0, sem1):
        wid = lax.axis_index("s") * NC + lax.axis_index("c")
        base = wid * PER_WS
        pltpu.sync_copy(idx_hbm.at[wid], idx_v)
        pltpu.async_copy(table_hbm.at[idx_v.at[0]], rows_v.at[0], sem0)

        def body(i, carry):
            c0 = i * 2
            c1 = c0 + 1
            pltpu.async_copy(table_hbm.at[idx_v.at[c1]], rows_v.at[1], sem1)
            pltpu.make_async_copy(
                table_hbm.at[idx_v.at[c0]], rows_v.at[0], sem0).wait()
            _pack_rows(rows_v.at[0], rowsb_v.at[0])
            pltpu.sync_copy(rowsb_v.at[0],
                            out_hbm.at[pl.ds(base + c0 * CHUNK, CHUNK)])

            @pl.when(c1 + 1 < NCH_S)
            def _():
                pltpu.async_copy(
                    table_hbm.at[idx_v.at[c1 + 1]], rows_v.at[0], sem0)

            pltpu.make_async_copy(
                table_hbm.at[idx_v.at[c1]], rows_v.at[1], sem1).wait()
            _pack_rows(rows_v.at[1], rowsb_v.at[1])
            pltpu.sync_copy(rowsb_v.at[1],
                            out_hbm.at[pl.ds(base + c1 * CHUNK, CHUNK)])
            return carry

        lax.fori_loop(0, NCH_S // 2, body, 0)

    return k(word_table, ids_grp)


_DN_T = (((0,), (0,)), ((), ()))
KCAT = 128 + 256 + 8


_DN_M = (((1,), (0,)), ((), ()))


def _tc_core(rows_ref, age_ref, seg_ref, pos_ref, catt_ref, jmat_ref,
             pm_ref, g_ref, b_ref, out_ref):
    x = lax.dot_general(rows_ref[...], pm_ref[...], _DN_M,
                        preferred_element_type=jnp.float32)
    age = age_ref[0]
    pos = pos_ref[0]
    seg = seg_ref[0]
    oh_a = (lax.broadcasted_iota(jnp.int32, (128, TBLK), 0) == age)
    oh_p = (lax.broadcasted_iota(jnp.int32, (256, TBLK), 0) == pos)
    oh_s = (lax.broadcasted_iota(jnp.int32, (8, TBLK), 0) == seg)
    oh = jnp.concatenate([oh_a, oh_p, oh_s], axis=0).astype(jnp.bfloat16)
    x = x + lax.dot_general(oh, catt_ref[...], _DN_T,
                            preferred_element_type=jnp.float32)
    jm = jmat_ref[...]
    mean = jnp.dot(x.astype(jnp.bfloat16), jm,
                   preferred_element_type=jnp.float32)
    msq = jnp.dot((x * x).astype(jnp.bfloat16), jm,
                  preferred_element_type=jnp.float32)
    var = msq - mean * mean
    inv = lax.rsqrt(var + 1e-12)
    out_ref[...] = (x - mean) * inv * g_ref[...] + b_ref[...]


def _tc_body(rows_ref, age_ref, seg_ref, pos_ref, catt_ref, jmat_ref,
             pm_ref, g_ref, b_ref, out_ref):
    _tc_core(rows_ref, age_ref, seg_ref, pos_ref, catt_ref, jmat_ref,
             pm_ref, g_ref, b_ref, out_ref)


def _tc_body_alias(prev_ref, rows_ref, age_ref, seg_ref, pos_ref, catt_ref,
                   jmat_ref, pm_ref, g_ref, b_ref, out_ref):
    del prev_ref
    _tc_core(rows_ref, age_ref, seg_ref, pos_ref, catt_ref, jmat_ref,
             pm_ref, g_ref, b_ref, out_ref)


def _tc_slice(s, prev_out, rows, age3, seg3, pos3, catt, jmat, pm,
              gamma2, beta2):
    data_specs = [
        pl.BlockSpec((TBLK, HIDDEN), lambda b: (b, 0)),
        pl.BlockSpec((1, 1, TBLK), lambda b: (b, 0, 0)),
        pl.BlockSpec((1, 1, TBLK), lambda b: (b, 0, 0)),
        pl.BlockSpec((1, 1, TBLK), lambda b: (b, 0, 0)),
        pl.BlockSpec((KCAT, HIDDEN), lambda b: (0, 0)),
        pl.BlockSpec((HIDDEN, HIDDEN), lambda b: (0, 0)),
        pl.BlockSpec((HIDDEN, HIDDEN), lambda b: (0, 0)),
        pl.BlockSpec((1, HIDDEN), lambda b: (0, 0)),
        pl.BlockSpec((1, HIDDEN), lambda b: (0, 0)),
    ]
    out_spec = pl.BlockSpec((TBLK, HIDDEN), lambda b, s=s: (b * S + s, 0))
    args = (rows, age3, seg3, pos3, catt, jmat, pm, gamma2, beta2)
    if prev_out is None:
        return pl.pallas_call(
            _tc_body,
            grid=(NW,),
            in_specs=data_specs,
            out_specs=out_spec,
            out_shape=jax.ShapeDtypeStruct((N, HIDDEN), jnp.float32),
            compiler_params=pltpu.CompilerParams(
                dimension_semantics=("arbitrary",)),
        )(*args)
    return pl.pallas_call(
        _tc_body_alias,
        grid=(NW,),
        in_specs=[pl.BlockSpec(memory_space=pl.ANY)] + data_specs,
        out_specs=out_spec,
        out_shape=jax.ShapeDtypeStruct((N, HIDDEN), jnp.float32),
        input_output_aliases={0: 0},
        compiler_params=pltpu.CompilerParams(
            dimension_semantics=("arbitrary",)),
    )(prev_out, *args)


def kernel(word_ids, age_ids, seg_ids, posi_ids, word_table, seg_table,
           age_table, posi_table, gamma, beta):
    ids_grp = word_ids.astype(jnp.int32).reshape(NW, S, NCH_S, CHUNK)
    age_g = age_ids.astype(jnp.int32).reshape(NW, S, 1, PER_WS)
    seg_g = seg_ids.astype(jnp.int32).reshape(NW, S, 1, PER_WS)
    pos_g = posi_ids.astype(jnp.int32).reshape(NW, S, 1, PER_WS)

    aget = jnp.pad(age_table, ((0, 128 - age_table.shape[0]), (0, 0)))
    segt = jnp.pad(seg_table, ((0, 8 - seg_table.shape[0]), (0, 0)))
    post = posi_table[:256]
    catt = jnp.concatenate([aget, post, segt], axis=0).astype(jnp.bfloat16)
    jmat = jnp.full((HIDDEN, HIDDEN), 1.0 / HIDDEN, dtype=jnp.bfloat16)
    pm = jnp.asarray(_PM_NP, dtype=jnp.bfloat16)
    gamma2 = gamma.reshape(1, HIDDEN)
    beta2 = beta.reshape(1, HIDDEN)

    rows = [
        lax.bitcast_convert_type(
            _sc_word_gather(word_table, ids_grp[:, s]), jnp.bfloat16
        ).reshape(NSL, HIDDEN)
        for s in range(S)
    ]
    out = None
    for s in range(S):
        out = _tc_slice(s, out, rows[s], age_g[:, s], seg_g[:, s],
                        pos_g[:, s], catt, jmat, pm, gamma2, beta2)
    return out.reshape(B, L, HIDDEN)


# speedup vs baseline: 3.0127x; 3.0127x over previous
import functools

import jax
import jax.numpy as jnp
import numpy as np
from jax import lax
from jax.experimental import pallas as pl
from jax.experimental.pallas import tpu as pltpu
from jax.experimental.pallas import tpu_sc as plsc

B, L, HIDDEN = 1024, 200, 128
N = B * L
NC, NS = 2, 16
NW = NC * NS
PER_W = N // NW
S = 5
PER_WS = PER_W // S
CHUNK = 128
NCH_S = PER_WS // CHUNK
NSL = NW * PER_WS
TBLK = PER_WS
NBLK_TOTAL = N // TBLK


def _pack_rows(rows_f32, rows_i32):
    def prow(r, carry):
        for p in range(4):
            a = lax.bitcast_convert_type(
                rows_f32[r, pl.ds(p * 32, 16)], jnp.int32) + 32768
            bb = lax.bitcast_convert_type(
                rows_f32[r, pl.ds(p * 32 + 16, 16)], jnp.int32) + 32768
            lo = lax.shift_right_logical(a, 16)
            hi = lax.bitwise_and(bb, jnp.int32(-65536))
            rows_i32[r, pl.ds(p * 16, 16)] = lax.bitwise_or(lo, hi)
        return carry

    lax.fori_loop(0, CHUNK, prow, 0)


_pm_nat = np.empty((128,), np.int32)
for _m in range(128):
    _mm = _m if _m < 64 else _m - 64
    _pm_nat[_m] = 32 * (_mm // 16) + (_mm % 16) + (16 if _m >= 64 else 0)
_PM_NP = (_pm_nat[:, None] == np.arange(128)[None, :]).astype(np.float32)


def _sc_word_gather(word_table, ids_grp):
    @functools.partial(
        pl.kernel,
        mesh=plsc.VectorSubcoreMesh(core_axis_name="c", subcore_axis_name="s"),
        out_type=jax.ShapeDtypeStruct((NSL, HIDDEN // 2), jnp.int32),
        scratch_types=[
            pltpu.VMEM((NCH_S, CHUNK), jnp.int32),
            pltpu.VMEM((2, CHUNK, HIDDEN), jnp.float32),
            pltpu.VMEM((2, CHUNK, HIDDEN // 2), jnp.int32),
            pltpu.SemaphoreType.DMA,
            pltpu.SemaphoreType.DMA,
        ],
    )
    def k(table_hbm, idx_hbm, out_hbm, idx_v, rows_v, rowsb_v, sem0, sem1):
        wid = lax.axis_index("s") * NC + lax.axis_index("c")
        base = wid * PER_WS
        pltpu.sync_copy(idx_hbm.at[wid], idx_v)
        pltpu.async_copy(table_hbm.at[idx_v.at[0]], rows_v.at[0], sem0)

        def body(i, carry):
            c0 = i * 2
            c1 = c0 + 1
            pltpu.async_copy(table_hbm.at[idx_v.at[c1]], rows_v.at[1], sem1)
            pltpu.make_async_copy(
                table_hbm.at[idx_v.at[c0]], rows_v.at[0], sem0).wait()
            _pack_rows(rows_v.at[0], rowsb_v.at[0])
            pltpu.sync_copy(rowsb_v.at[0],
                            out_hbm.at[pl.ds(base + c0 * CHUNK, CHUNK)])

            @pl.when(c1 + 1 < NCH_S)
            def _():
                pltpu.async_copy(
                    table_hbm.at[idx_v.at[c1 + 1]], rows_v.at[0], sem0)

            pltpu.make_async_copy(
                table_hbm.at[idx_v.at[c1]], rows_v.at[1], sem1).wait()
            _pack_rows(rows_v.at[1], rowsb_v.at[1])
            pltpu.sync_copy(rowsb_v.at[1],
                            out_hbm.at[pl.ds(base + c1 * CHUNK, CHUNK)])
            return carry

        lax.fori_loop(0, NCH_S // 2, body, 0)

    return k(word_table, ids_grp)


_DN_T = (((0,), (0,)), ((), ()))
KCAT = 128 + 256 + 8


_DN_M = (((1,), (0,)), ((), ()))


def _tc_core(rows_ref, age_ref, seg_ref, pos_ref, catt_ref, jmat_ref,
             pm_ref, g_ref, b_ref, out_ref):
    xi = rows_ref[...]
    af = lax.bitcast_convert_type(lax.shift_left(xi, 16), jnp.float32)
    bf = lax.bitcast_convert_type(
        lax.bitwise_and(xi, jnp.int32(-65536)), jnp.float32)
    xb = jnp.concatenate([af, bf], axis=1).astype(jnp.bfloat16)
    x = lax.dot_general(xb, pm_ref[...], _DN_M,
                        preferred_element_type=jnp.float32)
    age = age_ref[0]
    pos = pos_ref[0]
    seg = seg_ref[0]
    oh_a = (lax.broadcasted_iota(jnp.int32, (128, TBLK), 0) == age)
    oh_p = (lax.broadcasted_iota(jnp.int32, (256, TBLK), 0) == pos)
    oh_s = (lax.broadcasted_iota(jnp.int32, (8, TBLK), 0) == seg)
    oh = jnp.concatenate([oh_a, oh_p, oh_s], axis=0).astype(jnp.bfloat16)
    x = x + lax.dot_general(oh, catt_ref[...], _DN_T,
                            preferred_element_type=jnp.float32)
    jm = jmat_ref[...]
    mean = jnp.dot(x.astype(jnp.bfloat16), jm,
                   preferred_element_type=jnp.float32)
    msq = jnp.dot((x * x).astype(jnp.bfloat16), jm,
                  preferred_element_type=jnp.float32)
    var = msq - mean * mean
    inv = lax.rsqrt(var + 1e-12)
    out_ref[...] = (x - mean) * inv * g_ref[...] + b_ref[...]


def _tc_body(rows_ref, age_ref, seg_ref, pos_ref, catt_ref, jmat_ref,
             pm_ref, g_ref, b_ref, out_ref):
    _tc_core(rows_ref, age_ref, seg_ref, pos_ref, catt_ref, jmat_ref,
             pm_ref, g_ref, b_ref, out_ref)


def _tc_body_alias(prev_ref, rows_ref, age_ref, seg_ref, pos_ref, catt_ref,
                   jmat_ref, pm_ref, g_ref, b_ref, out_ref):
    del prev_ref
    _tc_core(rows_ref, age_ref, seg_ref, pos_ref, catt_ref, jmat_ref,
             pm_ref, g_ref, b_ref, out_ref)


def _tc_slice(s, prev_out, rows, age3, seg3, pos3, catt, jmat, pm,
              gamma2, beta2):
    data_specs = [
        pl.BlockSpec((TBLK, HIDDEN // 2), lambda b: (b, 0)),
        pl.BlockSpec((1, 1, TBLK), lambda b: (b, 0, 0)),
        pl.BlockSpec((1, 1, TBLK), lambda b: (b, 0, 0)),
        pl.BlockSpec((1, 1, TBLK), lambda b: (b, 0, 0)),
        pl.BlockSpec((KCAT, HIDDEN), lambda b: (0, 0)),
        pl.BlockSpec((HIDDEN, HIDDEN), lambda b: (0, 0)),
        pl.BlockSpec((HIDDEN, HIDDEN), lambda b: (0, 0)),
        pl.BlockSpec((1, HIDDEN), lambda b: (0, 0)),
        pl.BlockSpec((1, HIDDEN), lambda b: (0, 0)),
    ]
    out_spec = pl.BlockSpec((TBLK, HIDDEN), lambda b, s=s: (b * S + s, 0))
    args = (rows, age3, seg3, pos3, catt, jmat, pm, gamma2, beta2)
    if prev_out is None:
        return pl.pallas_call(
            _tc_body,
            grid=(NW,),
            in_specs=data_specs,
            out_specs=out_spec,
            out_shape=jax.ShapeDtypeStruct((N, HIDDEN), jnp.float32),
            compiler_params=pltpu.CompilerParams(
                dimension_semantics=("arbitrary",)),
        )(*args)
    return pl.pallas_call(
        _tc_body_alias,
        grid=(NW,),
        in_specs=[pl.BlockSpec(memory_space=pl.ANY)] + data_specs,
        out_specs=out_spec,
        out_shape=jax.ShapeDtypeStruct((N, HIDDEN), jnp.float32),
        input_output_aliases={0: 0},
        compiler_params=pltpu.CompilerParams(
            dimension_semantics=("arbitrary",)),
    )(prev_out, *args)


def kernel(word_ids, age_ids, seg_ids, posi_ids, word_table, seg_table,
           age_table, posi_table, gamma, beta):
    ids_grp = word_ids.astype(jnp.int32).reshape(NW, S, NCH_S, CHUNK)
    age_g = age_ids.astype(jnp.int32).reshape(NW, S, 1, PER_WS)
    seg_g = seg_ids.astype(jnp.int32).reshape(NW, S, 1, PER_WS)
    pos_g = posi_ids.astype(jnp.int32).reshape(NW, S, 1, PER_WS)

    aget = jnp.pad(age_table, ((0, 128 - age_table.shape[0]), (0, 0)))
    segt = jnp.pad(seg_table, ((0, 8 - seg_table.shape[0]), (0, 0)))
    post = posi_table[:256]
    catt = jnp.concatenate([aget, post, segt], axis=0).astype(jnp.bfloat16)
    jmat = jnp.full((HIDDEN, HIDDEN), 1.0 / HIDDEN, dtype=jnp.bfloat16)
    pm = jnp.asarray(_PM_NP, dtype=jnp.bfloat16)
    gamma2 = gamma.reshape(1, HIDDEN)
    beta2 = beta.reshape(1, HIDDEN)

    rows = [_sc_word_gather(word_table, ids_grp[:, s]) for s in range(S)]
    out = None
    for s in range(S):
        out = _tc_slice(s, out, rows[s], age_g[:, s], seg_g[:, s],
                        pos_g[:, s], catt, jmat, pm, gamma2, beta2)
    return out.reshape(B, L, HIDDEN)


# speedup vs baseline: 3.3144x; 1.1001x over previous
import functools

import jax
import jax.numpy as jnp
from jax import lax
from jax.experimental import pallas as pl
from jax.experimental.pallas import tpu as pltpu
from jax.experimental.pallas import tpu_sc as plsc

B, L, HIDDEN = 1024, 200, 128
N = B * L
NC, NS = 2, 16
NW = NC * NS
PER_W = N // NW
S = 5
PER_WS = PER_W // S
CHUNK = 128
NCH_S = PER_WS // CHUNK
NSL = NW * PER_WS
TBLK = PER_WS
NBLK_TOTAL = N // TBLK


def _sc_word_gather(word_table, ids_grp):
    @functools.partial(
        pl.kernel,
        mesh=plsc.VectorSubcoreMesh(core_axis_name="c", subcore_axis_name="s"),
        out_type=jax.ShapeDtypeStruct((NSL, HIDDEN), jnp.float32),
        scratch_types=[
            pltpu.VMEM((NCH_S, CHUNK), jnp.int32),
            pltpu.VMEM((2, CHUNK, HIDDEN), jnp.float32),
            pltpu.SemaphoreType.DMA,
            pltpu.SemaphoreType.DMA,
        ],
    )
    def k(table_hbm, idx_hbm, out_hbm, idx_v, rows_v, sem0, sem1):
        wid = lax.axis_index("s") * NC + lax.axis_index("c")
        base = wid * PER_WS
        pltpu.sync_copy(idx_hbm.at[wid], idx_v)
        pltpu.async_copy(table_hbm.at[idx_v.at[0]], rows_v.at[0], sem0)

        def body(i, carry):
            c0 = i * 2
            c1 = c0 + 1
            pltpu.async_copy(table_hbm.at[idx_v.at[c1]], rows_v.at[1], sem1)
            pltpu.make_async_copy(
                table_hbm.at[idx_v.at[c0]], rows_v.at[0], sem0).wait()
            pltpu.sync_copy(rows_v.at[0],
                            out_hbm.at[pl.ds(base + c0 * CHUNK, CHUNK)])

            @pl.when(c1 + 1 < NCH_S)
            def _():
                pltpu.async_copy(
                    table_hbm.at[idx_v.at[c1 + 1]], rows_v.at[0], sem0)

            pltpu.make_async_copy(
                table_hbm.at[idx_v.at[c1]], rows_v.at[1], sem1).wait()
            pltpu.sync_copy(rows_v.at[1],
                            out_hbm.at[pl.ds(base + c1 * CHUNK, CHUNK)])
            return carry

        lax.fori_loop(0, NCH_S // 2, body, 0)

    return k(word_table, ids_grp)


_DN_T = (((0,), (0,)), ((), ()))
KCAT = 128 + 256 + 8


def _tc_core(rows_ref, age_ref, seg_ref, pos_ref, catt_ref, jmat_ref,
             g_ref, b_ref, out_ref):
    x = rows_ref[...]
    age = age_ref[0]
    pos = pos_ref[0]
    seg = seg_ref[0]
    oh_a = (lax.broadcasted_iota(jnp.int32, (128, TBLK), 0) == age)
    oh_p = (lax.broadcasted_iota(jnp.int32, (256, TBLK), 0) == pos)
    oh_s = (lax.broadcasted_iota(jnp.int32, (8, TBLK), 0) == seg)
    oh = jnp.concatenate([oh_a, oh_p, oh_s], axis=0).astype(jnp.bfloat16)
    x = x + lax.dot_general(oh, catt_ref[...], _DN_T,
                            preferred_element_type=jnp.float32)
    jm = jmat_ref[...]
    mean = jnp.dot(x.astype(jnp.bfloat16), jm,
                   preferred_element_type=jnp.float32)
    msq = jnp.dot((x * x).astype(jnp.bfloat16), jm,
                  preferred_element_type=jnp.float32)
    var = msq - mean * mean
    inv = lax.rsqrt(var + 1e-12)
    out_ref[...] = (x - mean) * inv * g_ref[...] + b_ref[...]


def _tc_body(rows_ref, age_ref, seg_ref, pos_ref, catt_ref, jmat_ref,
             g_ref, b_ref, out_ref):
    _tc_core(rows_ref, age_ref, seg_ref, pos_ref, catt_ref, jmat_ref,
             g_ref, b_ref, out_ref)


def _tc_body_alias(prev_ref, rows_ref, age_ref, seg_ref, pos_ref, catt_ref,
                   jmat_ref, g_ref, b_ref, out_ref):
    del prev_ref
    _tc_core(rows_ref, age_ref, seg_ref, pos_ref, catt_ref, jmat_ref,
             g_ref, b_ref, out_ref)


def _tc_slice(s, prev_out, rows, age3, seg3, pos3, catt, jmat,
              gamma2, beta2):
    data_specs = [
        pl.BlockSpec((TBLK, HIDDEN), lambda b: (b, 0)),
        pl.BlockSpec((1, 1, TBLK), lambda b: (b, 0, 0)),
        pl.BlockSpec((1, 1, TBLK), lambda b: (b, 0, 0)),
        pl.BlockSpec((1, 1, TBLK), lambda b: (b, 0, 0)),
        pl.BlockSpec((KCAT, HIDDEN), lambda b: (0, 0)),
        pl.BlockSpec((HIDDEN, HIDDEN), lambda b: (0, 0)),
        pl.BlockSpec((1, HIDDEN), lambda b: (0, 0)),
        pl.BlockSpec((1, HIDDEN), lambda b: (0, 0)),
    ]
    out_spec = pl.BlockSpec((TBLK, HIDDEN), lambda b, s=s: (b * S + s, 0))
    args = (rows, age3, seg3, pos3, catt, jmat, gamma2, beta2)
    if prev_out is None:
        return pl.pallas_call(
            _tc_body,
            grid=(NW,),
            in_specs=data_specs,
            out_specs=out_spec,
            out_shape=jax.ShapeDtypeStruct((N, HIDDEN), jnp.float32),
            compiler_params=pltpu.CompilerParams(
                dimension_semantics=("arbitrary",)),
        )(*args)
    return pl.pallas_call(
        _tc_body_alias,
        grid=(NW,),
        in_specs=[pl.BlockSpec(memory_space=pl.ANY)] + data_specs,
        out_specs=out_spec,
        out_shape=jax.ShapeDtypeStruct((N, HIDDEN), jnp.float32),
        input_output_aliases={0: 0},
        compiler_params=pltpu.CompilerParams(
            dimension_semantics=("arbitrary",)),
    )(prev_out, *args)


def kernel(word_ids, age_ids, seg_ids, posi_ids, word_table, seg_table,
           age_table, posi_table, gamma, beta):
    ids_grp = word_ids.astype(jnp.int32).reshape(NW, S, NCH_S, CHUNK)
    age_g = age_ids.astype(jnp.int32).reshape(NW, S, 1, PER_WS)
    seg_g = seg_ids.astype(jnp.int32).reshape(NW, S, 1, PER_WS)
    pos_g = posi_ids.astype(jnp.int32).reshape(NW, S, 1, PER_WS)

    aget = jnp.pad(age_table, ((0, 128 - age_table.shape[0]), (0, 0)))
    segt = jnp.pad(seg_table, ((0, 8 - seg_table.shape[0]), (0, 0)))
    post = posi_table[:256]
    catt = jnp.concatenate([aget, post, segt], axis=0).astype(jnp.bfloat16)
    jmat = jnp.full((HIDDEN, HIDDEN), 1.0 / HIDDEN, dtype=jnp.bfloat16)
    gamma2 = gamma.reshape(1, HIDDEN)
    beta2 = beta.reshape(1, HIDDEN)

    rows = [_sc_word_gather(word_table, ids_grp[:, s]) for s in range(S)]
    out = None
    for s in range(S):
        out = _tc_slice(s, out, rows[s], age_g[:, s], seg_g[:, s],
                        pos_g[:, s], catt, jmat, gamma2, beta2)
    return out.reshape(B, L, HIDDEN)


# speedup vs baseline: 3.3263x; 1.0036x over previous
import functools

import jax
import jax.numpy as jnp
from jax import lax
from jax.experimental import pallas as pl
from jax.experimental.pallas import tpu as pltpu
from jax.experimental.pallas import tpu_sc as plsc

B, L, HIDDEN = 1024, 200, 128
N = B * L
NC, NS = 2, 16
NW = NC * NS
PER_W = N // NW
S = 5
PER_WS = PER_W // S
CHUNK = 128
NCH_S = PER_WS // CHUNK
NSL = NW * PER_WS
TBLK = PER_WS
NBLK_TOTAL = N // TBLK


def _sc_word_gather(word_table, ids_all, s):
    @functools.partial(
        pl.kernel,
        mesh=plsc.VectorSubcoreMesh(core_axis_name="c", subcore_axis_name="s"),
        out_type=jax.ShapeDtypeStruct((NSL, HIDDEN), jnp.float32),
        scratch_types=[
            pltpu.VMEM((NCH_S, CHUNK), jnp.int32),
            pltpu.VMEM((2, CHUNK, HIDDEN), jnp.float32),
            pltpu.SemaphoreType.DMA,
            pltpu.SemaphoreType.DMA,
        ],
    )
    def k(table_hbm, idx_hbm, out_hbm, idx_v, rows_v, sem0, sem1):
        wid = lax.axis_index("s") * NC + lax.axis_index("c")
        base = wid * PER_WS
        pltpu.sync_copy(idx_hbm.at[wid, s], idx_v)
        pltpu.async_copy(table_hbm.at[idx_v.at[0]], rows_v.at[0], sem0)

        def body(i, carry):
            c0 = i * 2
            c1 = c0 + 1
            pltpu.async_copy(table_hbm.at[idx_v.at[c1]], rows_v.at[1], sem1)
            pltpu.make_async_copy(
                table_hbm.at[idx_v.at[c0]], rows_v.at[0], sem0).wait()
            pltpu.sync_copy(rows_v.at[0],
                            out_hbm.at[pl.ds(base + c0 * CHUNK, CHUNK)])

            @pl.when(c1 + 1 < NCH_S)
            def _():
                pltpu.async_copy(
                    table_hbm.at[idx_v.at[c1 + 1]], rows_v.at[0], sem0)

            pltpu.make_async_copy(
                table_hbm.at[idx_v.at[c1]], rows_v.at[1], sem1).wait()
            pltpu.sync_copy(rows_v.at[1],
                            out_hbm.at[pl.ds(base + c1 * CHUNK, CHUNK)])
            return carry

        lax.fori_loop(0, NCH_S // 2, body, 0)

    return k(word_table, ids_all)


_DN_T = (((0,), (0,)), ((), ()))
KCAT = 128 + 256 + 8


def _tc_core(rows_ref, age_ref, seg_ref, pos_ref, catt_ref, jmat_ref,
             g_ref, b_ref, out_ref):
    x = rows_ref[...]
    age = age_ref[0]
    pos = pos_ref[0]
    seg = seg_ref[0]
    oh_a = (lax.broadcasted_iota(jnp.int32, (128, TBLK), 0) == age)
    oh_p = (lax.broadcasted_iota(jnp.int32, (256, TBLK), 0) == pos)
    oh_s = (lax.broadcasted_iota(jnp.int32, (8, TBLK), 0) == seg)
    oh = jnp.concatenate([oh_a, oh_p, oh_s], axis=0).astype(jnp.bfloat16)
    x = x + lax.dot_general(oh, catt_ref[...], _DN_T,
                            preferred_element_type=jnp.float32)
    jm = jmat_ref[...]
    mean = jnp.dot(x.astype(jnp.bfloat16), jm,
                   preferred_element_type=jnp.float32)
    msq = jnp.dot((x * x).astype(jnp.bfloat16), jm,
                  preferred_element_type=jnp.float32)
    var = msq - mean * mean
    inv = lax.rsqrt(var + 1e-12)
    out_ref[...] = (x - mean) * inv * g_ref[...] + b_ref[...]


def _tc_body(rows_ref, age_ref, seg_ref, pos_ref, catt_ref, jmat_ref,
             g_ref, b_ref, out_ref):
    _tc_core(rows_ref, age_ref, seg_ref, pos_ref, catt_ref, jmat_ref,
             g_ref, b_ref, out_ref)


def _tc_body_alias(prev_ref, rows_ref, age_ref, seg_ref, pos_ref, catt_ref,
                   jmat_ref, g_ref, b_ref, out_ref):
    del prev_ref
    _tc_core(rows_ref, age_ref, seg_ref, pos_ref, catt_ref, jmat_ref,
             g_ref, b_ref, out_ref)


def _tc_slice(s, prev_out, rows, age3, seg3, pos3, catt, jmat,
              gamma2, beta2):
    data_specs = [
        pl.BlockSpec((TBLK, HIDDEN), lambda b: (b, 0)),
        pl.BlockSpec((1, 1, TBLK), lambda b, s=s: (b * S + s, 0, 0)),
        pl.BlockSpec((1, 1, TBLK), lambda b, s=s: (b * S + s, 0, 0)),
        pl.BlockSpec((1, 1, TBLK), lambda b, s=s: (b * S + s, 0, 0)),
        pl.BlockSpec((KCAT, HIDDEN), lambda b: (0, 0)),
        pl.BlockSpec((HIDDEN, HIDDEN), lambda b: (0, 0)),
        pl.BlockSpec((1, HIDDEN), lambda b: (0, 0)),
        pl.BlockSpec((1, HIDDEN), lambda b: (0, 0)),
    ]
    out_spec = pl.BlockSpec((TBLK, HIDDEN), lambda b, s=s: (b * S + s, 0))
    args = (rows, age3, seg3, pos3, catt, jmat, gamma2, beta2)
    if prev_out is None:
        return pl.pallas_call(
            _tc_body,
            grid=(NW,),
            in_specs=data_specs,
            out_specs=out_spec,
            out_shape=jax.ShapeDtypeStruct((N, HIDDEN), jnp.float32),
            compiler_params=pltpu.CompilerParams(
                dimension_semantics=("arbitrary",)),
        )(*args)
    return pl.pallas_call(
        _tc_body_alias,
        grid=(NW,),
        in_specs=[pl.BlockSpec(memory_space=pl.ANY)] + data_specs,
        out_specs=out_spec,
        out_shape=jax.ShapeDtypeStruct((N, HIDDEN), jnp.float32),
        input_output_aliases={0: 0},
        compiler_params=pltpu.CompilerParams(
            dimension_semantics=("arbitrary",)),
    )(prev_out, *args)


def kernel(word_ids, age_ids, seg_ids, posi_ids, word_table, seg_table,
           age_table, posi_table, gamma, beta):
    ids_all = word_ids.astype(jnp.int32).reshape(NW, S, NCH_S, CHUNK)
    age_g = age_ids.astype(jnp.int32).reshape(NW * S, 1, PER_WS)
    seg_g = seg_ids.astype(jnp.int32).reshape(NW * S, 1, PER_WS)
    pos_g = posi_ids.astype(jnp.int32).reshape(NW * S, 1, PER_WS)

    aget = jnp.pad(age_table, ((0, 128 - age_table.shape[0]), (0, 0)))
    segt = jnp.pad(seg_table, ((0, 8 - seg_table.shape[0]), (0, 0)))
    post = posi_table[:256]
    catt = jnp.concatenate([aget, post, segt], axis=0).astype(jnp.bfloat16)
    jmat = jnp.full((HIDDEN, HIDDEN), 1.0 / HIDDEN, dtype=jnp.bfloat16)
    gamma2 = gamma.reshape(1, HIDDEN)
    beta2 = beta.reshape(1, HIDDEN)

    rows = [_sc_word_gather(word_table, ids_all, s) for s in range(S)]
    out = None
    for s in range(S):
        out = _tc_slice(s, out, rows[s], age_g, seg_g, pos_g,
                        catt, jmat, gamma2, beta2)
    return out.reshape(B, L, HIDDEN)
